# baseline (speedup 1.0000x reference)
import jax
import jax.numpy as jnp
from jax.experimental import pallas as pl
from jax.experimental.pallas import tpu as pltpu

_WORKER = 1000
_TASK = 20000
_ET = 2
_AB = 128
_K = _TASK * _ET
_WB = 40


def _body(b_ref, wf_ref, w_ref, tau_ref, out_ref):
    z = jnp.dot(wf_ref[...], w_ref[...],
                preferred_element_type=jnp.float32) + b_ref[0]
    s = jax.nn.sigmoid(z)
    lp1 = jnp.maximum(jnp.log(s), -1e30)[:, :, None]
    lp2 = jnp.maximum(jnp.log(1.0 - s), -1e30)[:, :, None]
    t = tau_ref[...]
    out_ref[...] = jnp.exp(t * lp1 + (1.0 - t) * lp2)


def kernel(inputs, W, b, P0):
    wf = inputs[:_WORKER]
    tau = inputs[_WORKER:, :_ET].T[None]
    out = pl.pallas_call(
        _body,
        grid=(_WORKER // _WB,),
        in_specs=[
            pl.BlockSpec(memory_space=pltpu.SMEM),
            pl.BlockSpec((_WB, _AB), lambda i: (i, 0)),
            pl.BlockSpec((_AB, 1), lambda i: (0, 0)),
            pl.BlockSpec((1, _ET, _TASK), lambda i: (0, 0, 0)),
        ],
        out_specs=pl.BlockSpec((_WB, _ET, _TASK), lambda i: (i, 0, 0)),
        out_shape=jax.ShapeDtypeStruct((_WORKER, _ET, _TASK), jnp.float32),
    )(b, wf, W, tau)
    return out.transpose(0, 2, 1)


# speedup vs baseline: 1.0610x; 1.0610x over previous
import jax
import jax.numpy as jnp
from jax.experimental import pallas as pl
from jax.experimental.pallas import tpu as pltpu

_WORKER = 1000
_TASK = 20000
_ET = 2
_AB = 128
_K = _TASK * _ET
_WB = 40


def _body(b_ref, wf_ref, w_ref, tau_ref, out_ref):
    z = jnp.dot(wf_ref[...], w_ref[...],
                preferred_element_type=jnp.float32) + b_ref[0]
    s = jax.nn.sigmoid(z)
    lp1 = jnp.maximum(jnp.log(s), -1e10)
    lp2 = jnp.maximum(jnp.log(1.0 - s), -1e10)
    d = (lp1 - lp2)[:, :, None]
    out_ref[...] = jnp.exp(lp2[:, :, None] + tau_ref[...] * d)


def kernel(inputs, W, b, P0):
    wf = inputs[:_WORKER]
    tau = inputs[_WORKER:, :_ET].T[None]
    out = pl.pallas_call(
        _body,
        grid=(_WORKER // _WB,),
        in_specs=[
            pl.BlockSpec(memory_space=pltpu.SMEM),
            pl.BlockSpec((_WB, _AB), lambda i: (i, 0)),
            pl.BlockSpec((_AB, 1), lambda i: (0, 0)),
            pl.BlockSpec((1, _ET, _TASK), lambda i: (0, 0, 0)),
        ],
        out_specs=pl.BlockSpec((_WB, _ET, _TASK), lambda i: (i, 0, 0)),
        out_shape=jax.ShapeDtypeStruct((_WORKER, _ET, _TASK), jnp.float32),
    )(b, wf, W, tau)
    return out.transpose(0, 2, 1)
